# baseline (speedup 1.0000x reference)
import functools

import jax
import jax.numpy as jnp
from jax import lax
from jax.experimental import pallas as pl
from jax.experimental.pallas import tpu as pltpu
from jax.experimental.pallas import tpu_sc as plsc

N = 10000
F = 128
NC, NS = 2, 16
NW = NC * NS
EB = 128
NBATCH = 80
E_PAD = NW * NBATCH * EB
NPAD = 10240
RPT = NPAD // NS
BR = 1024
GRID = NPAD // BR

def _mesh():
    return plsc.VectorSubcoreMesh(
        core_axis_name="c", subcore_axis_name="s",
        num_cores=NC, num_subcores=NS)


def _deg_body(dst_hbm, degp_hbm, dstv, degv):
    c = lax.axis_index("c")
    s = lax.axis_index("s")
    wid = c * NS + s
    pltpu.sync_copy(dst_hbm.at[wid], dstv)

    def zero(i, carry):
        degv[pl.ds(i * 16, 16)] = jnp.zeros((16,), jnp.float32)
        return carry
    lax.fori_loop(0, NPAD // 16, zero, 0)

    ones = jnp.ones((16,), jnp.float32)

    def acc(i, carry):
        idx = dstv[pl.ds(i * 16, 16)]
        plsc.addupdate_scatter(degv, [idx], ones)
        return carry
    lax.fori_loop(0, (NBATCH * EB) // 16, acc, 0)
    pltpu.sync_copy(degv, degp_hbm.at[wid])


def _sc_degree(dst2):
    return pl.kernel(
        _deg_body,
        out_type=jax.ShapeDtypeStruct((NW, NPAD), jnp.float32),
        mesh=_mesh(),
        compiler_params=pltpu.CompilerParams(needs_layout_passes=False),
        scratch_types=[
            pltpu.VMEM((NBATCH * EB,), jnp.int32),
            pltpu.VMEM((NPAD,), jnp.float32),
        ],
    )(dst2)


def _agg_body(table_hbm, src_hbm, dst_hbm, zeros_hbm, out_hbm,
              srcv, dstv, rows0, aggs, semg0):
    c = lax.axis_index("c")
    s = lax.axis_index("s")
    wid = c * NS + s
    pltpu.sync_copy(zeros_hbm, aggs.at[pl.ds(s * RPT, RPT)])
    pltpu.sync_copy(src_hbm.at[wid], srcv)
    pltpu.sync_copy(dst_hbm.at[wid], dstv)
    plsc.subcore_barrier()

    def step(j, carry):
        pltpu.async_copy(table_hbm.at[srcv.at[j]], rows0, semg0).wait()
        pltpu.sync_copy(rows0, aggs.at[dstv.at[j]], add=True)
        return carry
    lax.fori_loop(0, NBATCH, step, 0)
    plsc.subcore_barrier()
    pltpu.sync_copy(aggs.at[pl.ds(s * RPT, RPT)],
                    out_hbm.at[c, pl.ds(s * RPT, RPT)])


def _sc_aggregate(table, src3, dst3, zrows):
    return pl.kernel(
        _agg_body,
        out_type=jax.ShapeDtypeStruct((NC, NPAD, F), jnp.float32),
        mesh=_mesh(),
        scratch_types=[
            pltpu.VMEM((NBATCH, EB), jnp.int32),
            pltpu.VMEM((NBATCH, EB), jnp.int32),
            pltpu.VMEM((EB, F), jnp.float32),
            pltpu.VMEM_SHARED((NPAD, F), jnp.float32),
            pltpu.SemaphoreType.DMA,
        ],
    )(table, src3, dst3, zrows)


def _pass1_body(degp_ref, x_ref, xs_ref, dinv_ref):
    i = pl.program_id(0)
    deg = jnp.sum(degp_ref[...], axis=0) + 1.0
    row = lax.broadcasted_iota(jnp.int32, (BR,), 0) + i * BR
    dinv = jnp.where(row < N, lax.rsqrt(deg), 0.0)
    dinv_ref[...] = dinv
    xs_ref[...] = x_ref[...] * dinv[:, None]


def _tc_pass1(degp, xpad):
    return pl.pallas_call(
        _pass1_body,
        grid=(GRID,),
        in_specs=[
            pl.BlockSpec((NW, BR), lambda i: (0, i)),
            pl.BlockSpec((BR, F), lambda i: (i, 0)),
        ],
        out_specs=[
            pl.BlockSpec((BR, F), lambda i: (i, 0)),
            pl.BlockSpec((BR,), lambda i: (i,)),
        ],
        out_shape=[
            jax.ShapeDtypeStruct((NPAD, F), jnp.float32),
            jax.ShapeDtypeStruct((NPAD,), jnp.float32),
        ],
    )(degp, xpad)


def _pass2_body(a0_ref, a1_ref, xs_ref, dinv_ref, w1_ref, b1_ref, w2_ref,
                g2_ref):
    dinv = dinv_ref[...]
    sx = (a0_ref[...] + a1_ref[...] + xs_ref[...]) * dinv[:, None]
    h1 = jnp.dot(sx, w1_ref[...], preferred_element_type=jnp.float32)
    h1 = jnp.maximum(h1 + b1_ref[...][None, :], 0.0)
    g2 = jnp.dot(h1, w2_ref[...], preferred_element_type=jnp.float32)
    g2_ref[...] = g2 * dinv[:, None]


def _tc_pass2(a0, a1, xs, dinv, w1, b1, w2):
    d1, d2 = w1.shape[0], w1.shape[1]
    return pl.pallas_call(
        _pass2_body,
        grid=(GRID,),
        in_specs=[
            pl.BlockSpec((BR, F), lambda i: (i, 0)),
            pl.BlockSpec((BR, F), lambda i: (i, 0)),
            pl.BlockSpec((BR, F), lambda i: (i, 0)),
            pl.BlockSpec((BR,), lambda i: (i,)),
            pl.BlockSpec((d1, d2), lambda i: (0, 0)),
            pl.BlockSpec((d2,), lambda i: (0,)),
            pl.BlockSpec((d2, F), lambda i: (0, 0)),
        ],
        out_specs=pl.BlockSpec((BR, F), lambda i: (i, 0)),
        out_shape=jax.ShapeDtypeStruct((NPAD, F), jnp.float32),
    )(a0, a1, xs, dinv, w1, b1, w2)


def _pass3_body(a0_ref, a1_ref, g2_ref, dinv_ref, b2_ref, out_ref):
    dinv = dinv_ref[...]
    acc = (a0_ref[...] + a1_ref[...] + g2_ref[...]) * dinv[:, None]
    out_ref[...] = jnp.maximum(acc + b2_ref[...][None, :], 0.0)


def _tc_pass3(a0, a1, g2, dinv, b2):
    return pl.pallas_call(
        _pass3_body,
        grid=(GRID,),
        in_specs=[
            pl.BlockSpec((BR, F), lambda i: (i, 0)),
            pl.BlockSpec((BR, F), lambda i: (i, 0)),
            pl.BlockSpec((BR, F), lambda i: (i, 0)),
            pl.BlockSpec((BR,), lambda i: (i,)),
            pl.BlockSpec((F,), lambda i: (0,)),
        ],
        out_specs=pl.BlockSpec((BR, F), lambda i: (i, 0)),
        out_shape=jax.ShapeDtypeStruct((NPAD, F), jnp.float32),
    )(a0, a1, g2, dinv, b2)


@jax.jit
def _run(x, edge_index, W1, b1, W2, b2):
    src = edge_index[0].astype(jnp.int32)
    dst = edge_index[1].astype(jnp.int32)
    npad_e = E_PAD - src.shape[0]
    padv = jnp.full((npad_e,), N, jnp.int32)
    src_p = jnp.concatenate([src, padv])
    dst_p = jnp.concatenate([dst, padv])
    src3 = src_p.reshape(NW, NBATCH, EB)
    dst3 = dst_p.reshape(NW, NBATCH, EB)
    dst2 = dst_p.reshape(NW, NBATCH * EB)
    xpad = jnp.concatenate(
        [x.astype(jnp.float32), jnp.zeros((NPAD - N, F), jnp.float32)])
    zrows = jnp.zeros((RPT, F), jnp.float32)

    degp = _sc_degree(dst2)
    xs, dinv = _tc_pass1(degp, xpad)
    agg1 = _sc_aggregate(xs, src3, dst3, zrows)
    g2 = _tc_pass2(agg1[0], agg1[1], xs, dinv, W1, b1, W2)
    agg2 = _sc_aggregate(g2, src3, dst3, zrows)
    out = _tc_pass3(agg2[0], agg2[1], g2, dinv, b2)
    return out[:N]


def kernel(x, edge_index, W1, b1, W2, b2):
    return _run(x, edge_index, W1, b1, W2, b2)


# speedup vs baseline: 1.4423x; 1.4423x over previous
import functools

import jax
import jax.numpy as jnp
from jax import lax
from jax.experimental import pallas as pl
from jax.experimental.pallas import tpu as pltpu
from jax.experimental.pallas import tpu_sc as plsc

N = 10000
F = 128
NC, NS = 2, 16
NW = NC * NS
EB = 128
NBATCH = 79
E_PAD = NW * NBATCH * EB
NPAD = 10240
RPT = NPAD // NS
BR = 1024
GRID = NPAD // BR

def _mesh():
    return plsc.VectorSubcoreMesh(
        core_axis_name="c", subcore_axis_name="s",
        num_cores=NC, num_subcores=NS)


def _deg_body(dst_hbm, degp_hbm, dstv, degv):
    c = lax.axis_index("c")
    s = lax.axis_index("s")
    wid = c * NS + s
    pltpu.sync_copy(dst_hbm.at[wid], dstv)

    def zero(i, carry):
        degv[pl.ds(i * 16, 16)] = jnp.zeros((16,), jnp.float32)
        return carry
    lax.fori_loop(0, NPAD // 16, zero, 0)

    ones = jnp.ones((16,), jnp.float32)

    def acc(i, carry):
        idx = dstv[pl.ds(i * 16, 16)]
        plsc.addupdate_scatter(degv, [idx], ones)
        return carry
    lax.fori_loop(0, (NBATCH * EB) // 16, acc, 0)
    pltpu.sync_copy(degv, degp_hbm.at[wid])


def _sc_degree(dst2):
    return pl.kernel(
        _deg_body,
        out_type=jax.ShapeDtypeStruct((NW, NPAD), jnp.float32),
        mesh=_mesh(),
        compiler_params=pltpu.CompilerParams(needs_layout_passes=False),
        scratch_types=[
            pltpu.VMEM((NBATCH * EB,), jnp.int32),
            pltpu.VMEM((NPAD,), jnp.float32),
        ],
    )(dst2)


def _agg_body(table_hbm, src_hbm, dst_hbm, zeros_hbm, out_hbm,
              srcv, dstv, rows0, aggs, semg0):
    c = lax.axis_index("c")
    s = lax.axis_index("s")
    wid = c * NS + s
    pltpu.sync_copy(zeros_hbm, aggs.at[pl.ds(s * RPT, RPT)])
    pltpu.sync_copy(src_hbm.at[wid], srcv)
    pltpu.sync_copy(dst_hbm.at[wid], dstv)
    plsc.subcore_barrier()

    def step(j, carry):
        pltpu.async_copy(table_hbm.at[srcv.at[j]], rows0, semg0).wait()
        pltpu.sync_copy(rows0, aggs.at[dstv.at[j]], add=True)
        return carry
    lax.fori_loop(0, NBATCH, step, 0)
    plsc.subcore_barrier()
    pltpu.sync_copy(aggs.at[pl.ds(s * RPT, RPT)],
                    out_hbm.at[c, pl.ds(s * RPT, RPT)])


def _sc_aggregate(table, src3, dst3, zrows):
    return pl.kernel(
        _agg_body,
        out_type=jax.ShapeDtypeStruct((NC, NPAD, F), jnp.float32),
        mesh=_mesh(),
        scratch_types=[
            pltpu.VMEM((NBATCH, EB), jnp.int32),
            pltpu.VMEM((NBATCH, EB), jnp.int32),
            pltpu.VMEM((EB, F), jnp.float32),
            pltpu.VMEM_SHARED((NPAD, F), jnp.float32),
            pltpu.SemaphoreType.DMA,
        ],
    )(table, src3, dst3, zrows)


def _pass1_body(degp_ref, x_ref, xs_ref, dinv_ref):
    i = pl.program_id(0)
    deg = jnp.sum(degp_ref[...], axis=0) + 1.0
    row = lax.broadcasted_iota(jnp.int32, (BR,), 0) + i * BR
    dinv = jnp.where(row < N, lax.rsqrt(deg), 0.0)
    dinv_ref[...] = dinv
    xs_ref[...] = x_ref[...] * dinv[:, None]


def _tc_pass1(degp, xpad):
    return pl.pallas_call(
        _pass1_body,
        grid=(GRID,),
        in_specs=[
            pl.BlockSpec((NW, BR), lambda i: (0, i)),
            pl.BlockSpec((BR, F), lambda i: (i, 0)),
        ],
        out_specs=[
            pl.BlockSpec((BR, F), lambda i: (i, 0)),
            pl.BlockSpec((BR,), lambda i: (i,)),
        ],
        out_shape=[
            jax.ShapeDtypeStruct((NPAD, F), jnp.float32),
            jax.ShapeDtypeStruct((NPAD,), jnp.float32),
        ],
    )(degp, xpad)


def _pass2_body(a0_ref, a1_ref, xs_ref, dinv_ref, w1_ref, b1_ref, w2_ref,
                g2_ref):
    dinv = dinv_ref[...]
    sx = (a0_ref[...] + a1_ref[...] + xs_ref[...]) * dinv[:, None]
    h1 = jnp.dot(sx, w1_ref[...], preferred_element_type=jnp.float32)
    h1 = jnp.maximum(h1 + b1_ref[...][None, :], 0.0)
    g2 = jnp.dot(h1, w2_ref[...], preferred_element_type=jnp.float32)
    g2_ref[...] = g2 * dinv[:, None]


def _tc_pass2(a0, a1, xs, dinv, w1, b1, w2):
    d1, d2 = w1.shape[0], w1.shape[1]
    return pl.pallas_call(
        _pass2_body,
        grid=(GRID,),
        in_specs=[
            pl.BlockSpec((BR, F), lambda i: (i, 0)),
            pl.BlockSpec((BR, F), lambda i: (i, 0)),
            pl.BlockSpec((BR, F), lambda i: (i, 0)),
            pl.BlockSpec((BR,), lambda i: (i,)),
            pl.BlockSpec((d1, d2), lambda i: (0, 0)),
            pl.BlockSpec((d2,), lambda i: (0,)),
            pl.BlockSpec((d2, F), lambda i: (0, 0)),
        ],
        out_specs=pl.BlockSpec((BR, F), lambda i: (i, 0)),
        out_shape=jax.ShapeDtypeStruct((NPAD, F), jnp.float32),
    )(a0, a1, xs, dinv, w1, b1, w2)


def _pass3_body(a0_ref, a1_ref, g2_ref, dinv_ref, b2_ref, out_ref):
    dinv = dinv_ref[...]
    acc = (a0_ref[...] + a1_ref[...] + g2_ref[...]) * dinv[:, None]
    out_ref[...] = jnp.maximum(acc + b2_ref[...][None, :], 0.0)


def _tc_pass3(a0, a1, g2, dinv, b2):
    return pl.pallas_call(
        _pass3_body,
        grid=(GRID,),
        in_specs=[
            pl.BlockSpec((BR, F), lambda i: (i, 0)),
            pl.BlockSpec((BR, F), lambda i: (i, 0)),
            pl.BlockSpec((BR, F), lambda i: (i, 0)),
            pl.BlockSpec((BR,), lambda i: (i,)),
            pl.BlockSpec((F,), lambda i: (0,)),
        ],
        out_specs=pl.BlockSpec((BR, F), lambda i: (i, 0)),
        out_shape=jax.ShapeDtypeStruct((NPAD, F), jnp.float32),
    )(a0, a1, g2, dinv, b2)


@jax.jit
def _run(x, edge_index, W1, b1, W2, b2):
    src = edge_index[0].astype(jnp.int32)
    dst = edge_index[1].astype(jnp.int32)
    npad_e = E_PAD - src.shape[0]
    padv = jnp.full((npad_e,), N, jnp.int32)
    src_p = jnp.concatenate([src, padv])
    dst_p = jnp.concatenate([dst, padv])
    src3 = src_p.reshape(NW, NBATCH, EB)
    dst3 = dst_p.reshape(NW, NBATCH, EB)
    dst2 = dst_p.reshape(NW, NBATCH * EB)
    xpad = jnp.concatenate(
        [x.astype(jnp.float32), jnp.zeros((NPAD - N, F), jnp.float32)])
    zrows = jnp.zeros((RPT, F), jnp.float32)

    degp = _sc_degree(dst2)
    xs, dinv = _tc_pass1(degp, xpad)
    agg1 = _sc_aggregate(xs, src3, dst3, zrows)
    g2 = _tc_pass2(agg1[0], agg1[1], xs, dinv, W1, b1, W2)
    agg2 = _sc_aggregate(g2, src3, dst3, zrows)
    out = _tc_pass3(agg2[0], agg2[1], g2, dinv, b2)
    return out[:N]


def kernel(x, edge_index, W1, b1, W2, b2):
    return _run(x, edge_index, W1, b1, W2, b2)


# speedup vs baseline: 1.4480x; 1.0039x over previous
import functools

import jax
import jax.numpy as jnp
from jax import lax
from jax.experimental import pallas as pl
from jax.experimental.pallas import tpu as pltpu
from jax.experimental.pallas import tpu_sc as plsc

N = 10000
F = 128
NC, NS = 2, 16
NW = NC * NS
EB = 128
NBATCH = 79
E_PAD = NW * NBATCH * EB
NPAD = 10240
RPT = NPAD // NS
BR = 1024
GRID = NPAD // BR

def _mesh():
    return plsc.VectorSubcoreMesh(
        core_axis_name="c", subcore_axis_name="s",
        num_cores=NC, num_subcores=NS)


def _deg_body(dst_hbm, degp_hbm, dstv, degv):
    c = lax.axis_index("c")
    s = lax.axis_index("s")
    wid = c * NS + s
    pltpu.sync_copy(dst_hbm.at[wid], dstv)

    def zero(i, carry):
        degv[pl.ds(i * 16, 16)] = jnp.zeros((16,), jnp.float32)
        return carry
    lax.fori_loop(0, NPAD // 16, zero, 0)

    ones = jnp.ones((16,), jnp.float32)

    def acc(i, carry):
        idx = dstv[pl.ds(i * 16, 16)]
        plsc.addupdate_scatter(degv, [idx], ones)
        return carry
    lax.fori_loop(0, (NBATCH * EB) // 16, acc, 0)
    pltpu.sync_copy(degv, degp_hbm.at[wid])


def _sc_degree(dst2):
    return pl.kernel(
        _deg_body,
        out_type=jax.ShapeDtypeStruct((NW, NPAD), jnp.float32),
        mesh=_mesh(),
        compiler_params=pltpu.CompilerParams(needs_layout_passes=False),
        scratch_types=[
            pltpu.VMEM((NBATCH * EB,), jnp.int32),
            pltpu.VMEM((NPAD,), jnp.float32),
        ],
    )(dst2)


def _agg_body(table_hbm, src_hbm, dst_hbm, zeros_hbm, out_hbm,
              srcv, dstv, rows0, aggs, semg0):
    c = lax.axis_index("c")
    s = lax.axis_index("s")
    wid = c * NS + s
    pltpu.sync_copy(zeros_hbm, aggs.at[pl.ds(s * RPT, RPT)])
    pltpu.sync_copy(src_hbm.at[wid], srcv)
    pltpu.sync_copy(dst_hbm.at[wid], dstv)
    plsc.subcore_barrier()

    def step(j, carry):
        pltpu.async_copy(table_hbm.at[srcv.at[j]], rows0, semg0).wait()
        pltpu.sync_copy(rows0, aggs.at[dstv.at[j]], add=True)
        return carry
    lax.fori_loop(0, NBATCH, step, 0)
    plsc.subcore_barrier()
    pltpu.sync_copy(aggs.at[pl.ds(s * RPT, RPT)],
                    out_hbm.at[c, pl.ds(s * RPT, RPT)])


def _sc_aggregate(table, src3, dst3, zrows):
    return pl.kernel(
        _agg_body,
        out_type=jax.ShapeDtypeStruct((NC, NPAD, F), jnp.float32),
        mesh=_mesh(),
        scratch_types=[
            pltpu.VMEM((NBATCH, EB), jnp.int32),
            pltpu.VMEM((NBATCH, EB), jnp.int32),
            pltpu.VMEM((EB, F), jnp.float32),
            pltpu.VMEM_SHARED((NPAD, F), jnp.float32),
            pltpu.SemaphoreType.DMA,
        ],
    )(table, src3, dst3, zrows)


def _pass1_body(degp_ref, x_ref, xs_ref, dinv_ref):
    i = pl.program_id(0)
    deg = jnp.sum(degp_ref[...], axis=0) + 1.0
    row = lax.broadcasted_iota(jnp.int32, (BR,), 0) + i * BR
    dinv = jnp.where(row < N, lax.rsqrt(deg), 0.0)
    dinv_ref[...] = dinv
    xs_ref[...] = x_ref[...] * dinv[:, None]


def _tc_pass1(degp, xpad):
    return pl.pallas_call(
        _pass1_body,
        grid=(GRID,),
        in_specs=[
            pl.BlockSpec((NW, BR), lambda i: (0, i)),
            pl.BlockSpec((BR, F), lambda i: (i, 0)),
        ],
        out_specs=[
            pl.BlockSpec((BR, F), lambda i: (i, 0)),
            pl.BlockSpec((BR,), lambda i: (i,)),
        ],
        out_shape=[
            jax.ShapeDtypeStruct((NPAD, F), jnp.float32),
            jax.ShapeDtypeStruct((NPAD,), jnp.float32),
        ],
    )(degp, xpad)


def _pass2_body(a0_ref, a1_ref, xs_ref, dinv_ref, w1_ref, b1_ref, w2_ref,
                g2_ref):
    dinv = dinv_ref[...]
    sx = (a0_ref[...] + a1_ref[...] + xs_ref[...]) * dinv[:, None]
    h1 = jnp.dot(sx, w1_ref[...], preferred_element_type=jnp.float32)
    h1 = jnp.maximum(h1 + b1_ref[...][None, :], 0.0)
    g2 = jnp.dot(h1, w2_ref[...], preferred_element_type=jnp.float32)
    g2_ref[...] = g2 * dinv[:, None]


def _tc_pass2(a0, a1, xs, dinv, w1, b1, w2):
    d1, d2 = w1.shape[0], w1.shape[1]
    return pl.pallas_call(
        _pass2_body,
        grid=(GRID,),
        in_specs=[
            pl.BlockSpec((BR, F), lambda i: (i, 0)),
            pl.BlockSpec((BR, F), lambda i: (i, 0)),
            pl.BlockSpec((BR, F), lambda i: (i, 0)),
            pl.BlockSpec((BR,), lambda i: (i,)),
            pl.BlockSpec((d1, d2), lambda i: (0, 0)),
            pl.BlockSpec((d2,), lambda i: (0,)),
            pl.BlockSpec((d2, F), lambda i: (0, 0)),
        ],
        out_specs=pl.BlockSpec((BR, F), lambda i: (i, 0)),
        out_shape=jax.ShapeDtypeStruct((NPAD, F), jnp.float32),
    )(a0, a1, xs, dinv, w1, b1, w2)


def _pass3_body(a0_ref, a1_ref, g2_ref, dinv_ref, b2_ref, out_ref):
    dinv = dinv_ref[...]
    acc = (a0_ref[...] + a1_ref[...] + g2_ref[...]) * dinv[:, None]
    out_ref[...] = jnp.maximum(acc + b2_ref[...][None, :], 0.0)


def _tc_pass3(a0, a1, g2, dinv, b2):
    return pl.pallas_call(
        _pass3_body,
        grid=(GRID,),
        in_specs=[
            pl.BlockSpec((BR, F), lambda i: (i, 0)),
            pl.BlockSpec((BR, F), lambda i: (i, 0)),
            pl.BlockSpec((BR, F), lambda i: (i, 0)),
            pl.BlockSpec((BR,), lambda i: (i,)),
            pl.BlockSpec((F,), lambda i: (0,)),
        ],
        out_specs=pl.BlockSpec((BR, F), lambda i: (i, 0)),
        out_shape=jax.ShapeDtypeStruct((NPAD, F), jnp.float32),
    )(a0, a1, g2, dinv, b2)


@jax.jit
def _run(x, edge_index, W1, b1, W2, b2):
    src = edge_index[0].astype(jnp.int32)
    dst = edge_index[1].astype(jnp.int32)
    npad_e = E_PAD - src.shape[0]
    padv = jnp.full((npad_e,), N, jnp.int32)
    pad_dst = N + (jnp.arange(npad_e, dtype=jnp.int32) % (NPAD - N))
    src_p = jnp.concatenate([src, padv])
    dst_p = jnp.concatenate([dst, pad_dst])
    src3 = src_p.reshape(NW, NBATCH, EB)
    dst3 = dst_p.reshape(NW, NBATCH, EB)
    dst2 = dst_p.reshape(NW, NBATCH * EB)
    xpad = jnp.concatenate(
        [x.astype(jnp.float32), jnp.zeros((NPAD - N, F), jnp.float32)])
    zrows = jnp.zeros((RPT, F), jnp.float32)

    degp = _sc_degree(dst2)
    xs, dinv = _tc_pass1(degp, xpad)
    agg1 = _sc_aggregate(xs, src3, dst3, zrows)
    g2 = _tc_pass2(agg1[0], agg1[1], xs, dinv, W1, b1, W2)
    agg2 = _sc_aggregate(g2, src3, dst3, zrows)
    out = _tc_pass3(agg2[0], agg2[1], g2, dinv, b2)
    return out[:N]


def kernel(x, edge_index, W1, b1, W2, b2):
    return _run(x, edge_index, W1, b1, W2, b2)
